# baseline (speedup 1.0000x reference)
import jax
import jax.numpy as jnp
from jax import lax
from jax.experimental import pallas as pl
from jax.experimental.pallas import tpu as pltpu
from jax.experimental.pallas import tpu_sc as plsc

_N = 16384
_B = 512
_FEAT = 128
_EMB = 64
_MESH = 64
_OUT = 128
_E = 262144
_NB = _N // _B
_AUG = 80

_CH = 128
_NW = 32
_EPW = _E // _NW
_NCH = _EPW // _CH
_RPS = _N // 16

_HIGH = lax.Precision.HIGHEST


def _mine_body(f_ref, wp_ref, sp_ref, pos_ref, neg_ref, aug_ref):
    f = f_ref[...]
    g = lax.dot_general(f, f, (((1,), (1,)), ((), ())),
                        preferred_element_type=jnp.float32)
    sq = jnp.sum(f * f, axis=1)
    d2 = sq[:, None] + sq[None, :] - 2.0 * g
    dkey = jnp.sqrt(jnp.maximum(d2, 0.0))
    col = lax.broadcasted_iota(jnp.int32, (_B, _B), 1)
    row = lax.broadcasted_iota(jnp.int32, (_B, _B), 0)
    inf = jnp.float32(jnp.inf)

    def pick(dm, onehots, largest):
        m = (jnp.max if largest else jnp.min)(dm, axis=1, keepdims=True)
        first = jnp.min(jnp.where(dm == m, col, _B), axis=1, keepdims=True)
        oh = col == first
        onehots.append(oh)
        return jnp.where(oh, -inf if largest else inf, dm)

    pos_oh = []
    dm = jnp.where(col == row, inf, dkey)
    for _ in range(5):
        dm = pick(dm, pos_oh, largest=False)
    neg_oh = []
    dm = dkey
    for _ in range(5):
        dm = pick(dm, neg_oh, largest=True)
    neg_oh.reverse()

    emb = lax.dot_general(f, wp_ref[...], (((1,), (0,)), ((), ())),
                          preferred_element_type=jnp.float32)
    eps = jnp.float32(1e-6)

    h1 = emb.astype(jnp.bfloat16).astype(jnp.float32)
    r1 = emb - h1
    h2 = r1.astype(jnp.bfloat16).astype(jnp.float32)
    h3 = r1 - h2

    def cand_dist(oh):
        ohf = oh.astype(jnp.float32)

        def ext(part):
            return lax.dot_general(ohf, part, (((1,), (0,)), ((), ())),
                                   preferred_element_type=jnp.float32)
        c = (ext(h1) + ext(h2)) + ext(h3)
        dif = emb - c + eps
        return c, jnp.sqrt(jnp.sum(dif * dif, axis=1, keepdims=True))

    bc, bd = cand_dist(pos_oh[0])
    for oh in pos_oh[1:]:
        c, dd = cand_dist(oh)
        upd = dd > bd
        bd = jnp.where(upd, dd, bd)
        bc = jnp.where(upd, c, bc)
    pos_ref[...] = bc

    bc, bd = cand_dist(neg_oh[0])
    for oh in neg_oh[1:]:
        c, dd = cand_dist(oh)
        upd = dd < bd
        bd = jnp.where(upd, dd, bd)
        bc = jnp.where(upd, c, bc)
    neg_ref[...] = bc

    sp_ref[...] = emb
    aug_ref[...] = jnp.concatenate(
        [emb, jnp.ones((_B, 1), jnp.float32),
         jnp.zeros((_B, _AUG - _EMB - 1), jnp.float32)], axis=1)


def _mine(feats, wp):
    return pl.pallas_call(
        _mine_body,
        grid=(_NB,),
        in_specs=[
            pl.BlockSpec((_B, _FEAT), lambda i: (i, 0)),
            pl.BlockSpec((_FEAT, _EMB), lambda i: (0, 0)),
        ],
        out_specs=[
            pl.BlockSpec((_B, _EMB), lambda i: (i, 0)),
            pl.BlockSpec((_B, _EMB), lambda i: (i, 0)),
            pl.BlockSpec((_B, _EMB), lambda i: (i, 0)),
            pl.BlockSpec((_B, _AUG), lambda i: (i, 0)),
        ],
        out_shape=[
            jax.ShapeDtypeStruct((_N, _EMB), jnp.float32),
            jax.ShapeDtypeStruct((_N, _EMB), jnp.float32),
            jax.ShapeDtypeStruct((_N, _EMB), jnp.float32),
            jax.ShapeDtypeStruct((_N, _AUG), jnp.float32),
        ],
    )(feats, wp)


def _seg_body(table, src, dst, out, src_v, dst_v, rows, zbuf, acc, sem):
    cid = lax.axis_index("c")
    sid = lax.axis_index("s")
    wid = sid * 2 + cid

    def zrow(i, carry):
        for j in range(_AUG // 16):
            zbuf[i, pl.ds(j * 16, 16)] = jnp.zeros((16,), jnp.float32)
        return carry
    lax.fori_loop(0, 128, zrow, 0)
    for r in range(_RPS // 128):
        pltpu.sync_copy(zbuf, acc.at[pl.ds(sid * _RPS + r * 128, 128)])
    plsc.subcore_barrier()

    pltpu.sync_copy(src.at[pl.ds(wid * _NCH, _NCH)], src_v)
    pltpu.sync_copy(dst.at[pl.ds(wid * _NCH, _NCH)], dst_v)

    def chunk(c, carry):
        pltpu.async_copy(table.at[src_v.at[c]], rows, sem).wait()
        pltpu.sync_copy(rows, acc.at[dst_v.at[c]], add=True)
        return carry
    lax.fori_loop(0, _NCH, chunk, 0)

    plsc.subcore_barrier()
    pltpu.sync_copy(acc.at[pl.ds(sid * _RPS, _RPS)],
                    out.at[pl.ds(cid * _N + sid * _RPS, _RPS)])


def _segsum(aug, src2d, dst2d):
    k = pl.kernel(
        _seg_body,
        out_type=jax.ShapeDtypeStruct((2 * _N, _AUG), jnp.float32),
        mesh=plsc.VectorSubcoreMesh(core_axis_name="c", subcore_axis_name="s"),
        scratch_types=[
            pltpu.VMEM((_NCH, _CH), jnp.int32),
            pltpu.VMEM((_NCH, _CH), jnp.int32),
            pltpu.VMEM((_CH, _AUG), jnp.float32),
            pltpu.VMEM((128, _AUG), jnp.float32),
            pltpu.VMEM_SHARED((_N, _AUG), jnp.float32),
            pltpu.SemaphoreType.DMA,
        ],
        compiler_params=pltpu.CompilerParams(use_tc_tiling_on_sc=False),
    )
    return k(aug, src2d, dst2d)


def _head_body(p0_ref, p1_ref, wm_ref, wc_ref, mg_ref, lg_ref):
    accv = p0_ref[...] + p1_ref[...]
    agg = accv[:, :_EMB]
    deg = accv[:, _EMB:_EMB + 1]
    mean = agg / jnp.maximum(deg, 1.0)
    mg = jnp.maximum(
        lax.dot_general(mean, wm_ref[...], (((1,), (0,)), ((), ())),
                        preferred_element_type=jnp.float32), 0.0)
    mg_ref[...] = mg
    lg_ref[...] = lax.dot_general(mg, wc_ref[...], (((1,), (0,)), ((), ())),
                                  preferred_element_type=jnp.float32)


def _head(partials, wm, wc):
    return pl.pallas_call(
        _head_body,
        grid=(_NB,),
        in_specs=[
            pl.BlockSpec((_B, _AUG), lambda i: (i, 0)),
            pl.BlockSpec((_B, _AUG), lambda i: (i + _NB, 0)),
            pl.BlockSpec((_MESH, _MESH), lambda i: (0, 0)),
            pl.BlockSpec((_MESH, _OUT), lambda i: (0, 0)),
        ],
        out_specs=[
            pl.BlockSpec((_B, _MESH), lambda i: (i, 0)),
            pl.BlockSpec((_B, _OUT), lambda i: (i, 0)),
        ],
        out_shape=[
            jax.ShapeDtypeStruct((_N, _MESH), jnp.float32),
            jax.ShapeDtypeStruct((_N, _OUT), jnp.float32),
        ],
    )(partials, partials, wm, wc)


def kernel(patch_feats, edge_index, W_patch, W_mesh, W_cls):
    sp, pos, neg, aug = _mine(patch_feats, W_patch)
    src2d = edge_index[0].reshape(_E // _CH, _CH)
    dst2d = edge_index[1].reshape(_E // _CH, _CH)
    partials = _segsum(aug, src2d, dst2d)
    mg, logits = _head(partials, W_mesh, W_cls)
    return (logits, mg, sp, pos, neg)


# speedup vs baseline: 1.1228x; 1.1228x over previous
import jax
import jax.numpy as jnp
from jax import lax
from jax.experimental import pallas as pl
from jax.experimental.pallas import tpu as pltpu
from jax.experimental.pallas import tpu_sc as plsc

_N = 16384
_B = 512
_FEAT = 128
_EMB = 64
_MESH = 64
_OUT = 128
_E = 262144
_NB = _N // _B
_AUG = 80

_CH = 128
_NW = 32
_EPW = _E // _NW
_NCH = _EPW // _CH
_RPS = _N // 16

_HIGH = lax.Precision.HIGHEST


def _mine_body(f_ref, wp_ref, sp_ref, pos_ref, neg_ref, aug_ref):
    f = f_ref[...]
    g = lax.dot_general(f, f, (((1,), (1,)), ((), ())),
                        preferred_element_type=jnp.float32)
    sq = jnp.sum(f * f, axis=1)
    d2 = sq[:, None] + sq[None, :] - 2.0 * g
    dkey = jnp.sqrt(jnp.maximum(d2, 0.0))
    col = lax.broadcasted_iota(jnp.int32, (_B, _B), 1)
    row = lax.broadcasted_iota(jnp.int32, (_B, _B), 0)
    inf = jnp.float32(jnp.inf)

    def pick(dm, onehots, largest):
        m = (jnp.max if largest else jnp.min)(dm, axis=1, keepdims=True)
        first = jnp.min(jnp.where(dm == m, col, _B), axis=1, keepdims=True)
        oh = col == first
        onehots.append(oh)
        return jnp.where(oh, -inf if largest else inf, dm)

    pos_oh = []
    dm = jnp.where(col == row, inf, dkey)
    for _ in range(5):
        dm = pick(dm, pos_oh, largest=False)
    neg_oh = []
    dm = dkey
    for _ in range(5):
        dm = pick(dm, neg_oh, largest=True)
    neg_oh.reverse()

    emb = lax.dot_general(f, wp_ref[...], (((1,), (0,)), ((), ())),
                          preferred_element_type=jnp.float32)
    eps = jnp.float32(1e-6)

    h1 = emb.astype(jnp.bfloat16).astype(jnp.float32)
    r1 = emb - h1
    h2 = r1.astype(jnp.bfloat16).astype(jnp.float32)
    h3 = r1 - h2

    def cand_dist(oh):
        ohf = oh.astype(jnp.float32)

        def ext(part):
            return lax.dot_general(ohf, part, (((1,), (0,)), ((), ())),
                                   preferred_element_type=jnp.float32)
        c = (ext(h1) + ext(h2)) + ext(h3)
        dif = emb - c + eps
        return c, jnp.sqrt(jnp.sum(dif * dif, axis=1, keepdims=True))

    bc, bd = cand_dist(pos_oh[0])
    for oh in pos_oh[1:]:
        c, dd = cand_dist(oh)
        upd = dd > bd
        bd = jnp.where(upd, dd, bd)
        bc = jnp.where(upd, c, bc)
    pos_ref[...] = bc

    bc, bd = cand_dist(neg_oh[0])
    for oh in neg_oh[1:]:
        c, dd = cand_dist(oh)
        upd = dd < bd
        bd = jnp.where(upd, dd, bd)
        bc = jnp.where(upd, c, bc)
    neg_ref[...] = bc

    sp_ref[...] = emb
    aug_ref[...] = jnp.concatenate(
        [emb, jnp.ones((_B, 1), jnp.float32),
         jnp.zeros((_B, _AUG - _EMB - 1), jnp.float32)], axis=1)


def _mine(feats, wp):
    return pl.pallas_call(
        _mine_body,
        grid=(_NB,),
        in_specs=[
            pl.BlockSpec((_B, _FEAT), lambda i: (i, 0)),
            pl.BlockSpec((_FEAT, _EMB), lambda i: (0, 0)),
        ],
        out_specs=[
            pl.BlockSpec((_B, _EMB), lambda i: (i, 0)),
            pl.BlockSpec((_B, _EMB), lambda i: (i, 0)),
            pl.BlockSpec((_B, _EMB), lambda i: (i, 0)),
            pl.BlockSpec((_B, _AUG), lambda i: (i, 0)),
        ],
        out_shape=[
            jax.ShapeDtypeStruct((_N, _EMB), jnp.float32),
            jax.ShapeDtypeStruct((_N, _EMB), jnp.float32),
            jax.ShapeDtypeStruct((_N, _EMB), jnp.float32),
            jax.ShapeDtypeStruct((_N, _AUG), jnp.float32),
        ],
    )(feats, wp)


def _seg_body(table, ei3, out, src_v, dst_v, rows0, rows1, zbuf, acc,
              sem0, sem1):
    cid = lax.axis_index("c")
    sid = lax.axis_index("s")
    wid = sid * 2 + cid

    def zrow(i, carry):
        for j in range(_AUG // 16):
            zbuf[i, pl.ds(j * 16, 16)] = jnp.zeros((16,), jnp.float32)
        return carry
    lax.fori_loop(0, 128, zrow, 0)
    for r in range(_RPS // 128):
        pltpu.sync_copy(zbuf, acc.at[pl.ds(sid * _RPS + r * 128, 128)])
    plsc.subcore_barrier()

    pltpu.sync_copy(ei3.at[0, pl.ds(wid * _NCH, _NCH)], src_v)
    pltpu.sync_copy(ei3.at[1, pl.ds(wid * _NCH, _NCH)], dst_v)

    def gather(c, buf, sem):
        return pltpu.make_async_copy(table.at[src_v.at[c]], buf, sem)

    gather(0, rows0, sem0).start()
    gather(1, rows1, sem1).start()

    def pair(j, carry):
        c0 = j * 2
        for c, buf, sem in ((c0, rows0, sem0), (c0 + 1, rows1, sem1)):
            gather(c, buf, sem).wait()
            pltpu.sync_copy(buf, acc.at[dst_v.at[c]], add=True)

            @pl.when(c + 2 < _NCH)
            def _():
                gather(c + 2, buf, sem).start()
        return carry
    lax.fori_loop(0, _NCH // 2, pair, 0)

    plsc.subcore_barrier()
    pltpu.sync_copy(acc.at[pl.ds(sid * _RPS, _RPS)],
                    out.at[pl.ds(cid * _N + sid * _RPS, _RPS)])


def _segsum(aug, ei3):
    k = pl.kernel(
        _seg_body,
        out_type=jax.ShapeDtypeStruct((2 * _N, _AUG), jnp.float32),
        mesh=plsc.VectorSubcoreMesh(core_axis_name="c", subcore_axis_name="s"),
        scratch_types=[
            pltpu.VMEM((_NCH, _CH), jnp.int32),
            pltpu.VMEM((_NCH, _CH), jnp.int32),
            pltpu.VMEM((_CH, _AUG), jnp.float32),
            pltpu.VMEM((_CH, _AUG), jnp.float32),
            pltpu.VMEM((128, _AUG), jnp.float32),
            pltpu.VMEM_SHARED((_N, _AUG), jnp.float32),
            pltpu.SemaphoreType.DMA,
            pltpu.SemaphoreType.DMA,
        ],
        compiler_params=pltpu.CompilerParams(use_tc_tiling_on_sc=False),
    )
    return k(aug, ei3)


def _head_body(p0_ref, p1_ref, wm_ref, wc_ref, mg_ref, lg_ref):
    accv = p0_ref[...] + p1_ref[...]
    agg = accv[:, :_EMB]
    deg = accv[:, _EMB:_EMB + 1]
    mean = agg / jnp.maximum(deg, 1.0)
    mg = jnp.maximum(
        lax.dot_general(mean, wm_ref[...], (((1,), (0,)), ((), ())),
                        preferred_element_type=jnp.float32), 0.0)
    mg_ref[...] = mg
    lg_ref[...] = lax.dot_general(mg, wc_ref[...], (((1,), (0,)), ((), ())),
                                  preferred_element_type=jnp.float32)


def _head(partials, wm, wc):
    return pl.pallas_call(
        _head_body,
        grid=(_NB,),
        in_specs=[
            pl.BlockSpec((_B, _AUG), lambda i: (i, 0)),
            pl.BlockSpec((_B, _AUG), lambda i: (i + _NB, 0)),
            pl.BlockSpec((_MESH, _MESH), lambda i: (0, 0)),
            pl.BlockSpec((_MESH, _OUT), lambda i: (0, 0)),
        ],
        out_specs=[
            pl.BlockSpec((_B, _MESH), lambda i: (i, 0)),
            pl.BlockSpec((_B, _OUT), lambda i: (i, 0)),
        ],
        out_shape=[
            jax.ShapeDtypeStruct((_N, _MESH), jnp.float32),
            jax.ShapeDtypeStruct((_N, _OUT), jnp.float32),
        ],
    )(partials, partials, wm, wc)


def kernel(patch_feats, edge_index, W_patch, W_mesh, W_cls):
    sp, pos, neg, aug = _mine(patch_feats, W_patch)
    partials = _segsum(aug, edge_index.reshape(2, _E // _CH, _CH))
    mg, logits = _head(partials, W_mesh, W_cls)
    return (logits, mg, sp, pos, neg)


# speedup vs baseline: 1.1232x; 1.0003x over previous
import jax
import jax.numpy as jnp
from jax import lax
from jax.experimental import pallas as pl
from jax.experimental.pallas import tpu as pltpu
from jax.experimental.pallas import tpu_sc as plsc

_N = 16384
_B = 512
_FEAT = 128
_EMB = 64
_MESH = 64
_OUT = 128
_E = 262144
_NB = _N // _B
_AUG = 80

_CH = 128
_NW = 32
_EPW = _E // _NW
_NCH = _EPW // _CH
_RPS = _N // 16

_HIGH = lax.Precision.HIGHEST


def _mine_body(f_ref, wp_ref, sp_ref, pos_ref, neg_ref, aug_ref):
    f = f_ref[...]
    g = lax.dot_general(f, f, (((1,), (1,)), ((), ())),
                        preferred_element_type=jnp.float32)
    sq = jnp.sum(f * f, axis=1)
    d2 = sq[:, None] + sq[None, :] - 2.0 * g
    dkey = jnp.sqrt(jnp.maximum(d2, 0.0))
    col = lax.broadcasted_iota(jnp.int32, (_B, _B), 1)
    row = lax.broadcasted_iota(jnp.int32, (_B, _B), 0)
    inf = jnp.float32(jnp.inf)

    def pick(dm, onehots, largest, last):
        m = (jnp.max if largest else jnp.min)(dm, axis=1, keepdims=True)
        first = jnp.min(jnp.where(dm == m, col, _B), axis=1, keepdims=True)
        oh = col == first
        onehots.append(oh)
        if last:
            return dm
        return jnp.where(oh, -inf if largest else inf, dm)

    pos_oh = []
    dm = jnp.where(col == row, inf, dkey)
    for k in range(5):
        dm = pick(dm, pos_oh, largest=False, last=k == 4)
    neg_oh = []
    dm = dkey
    for k in range(5):
        dm = pick(dm, neg_oh, largest=True, last=k == 4)
    neg_oh.reverse()

    emb = lax.dot_general(f, wp_ref[...], (((1,), (0,)), ((), ())),
                          preferred_element_type=jnp.float32)
    eps = jnp.float32(1e-6)

    h1 = emb.astype(jnp.bfloat16).astype(jnp.float32)
    r1 = emb - h1
    h2 = r1.astype(jnp.bfloat16).astype(jnp.float32)
    h3 = r1 - h2

    def cand_dist(oh):
        ohf = oh.astype(jnp.float32)

        def ext(part):
            return lax.dot_general(ohf, part, (((1,), (0,)), ((), ())),
                                   preferred_element_type=jnp.float32)
        c = (ext(h1) + ext(h2)) + ext(h3)
        dif = emb - c + eps
        return c, jnp.sqrt(jnp.sum(dif * dif, axis=1, keepdims=True))

    bc, bd = cand_dist(pos_oh[0])
    for oh in pos_oh[1:]:
        c, dd = cand_dist(oh)
        upd = dd > bd
        bd = jnp.where(upd, dd, bd)
        bc = jnp.where(upd, c, bc)
    pos_ref[...] = bc

    bc, bd = cand_dist(neg_oh[0])
    for oh in neg_oh[1:]:
        c, dd = cand_dist(oh)
        upd = dd < bd
        bd = jnp.where(upd, dd, bd)
        bc = jnp.where(upd, c, bc)
    neg_ref[...] = bc

    sp_ref[...] = emb
    aug_ref[...] = jnp.concatenate(
        [emb, jnp.ones((_B, 1), jnp.float32),
         jnp.zeros((_B, _AUG - _EMB - 1), jnp.float32)], axis=1)


def _mine(feats, wp):
    return pl.pallas_call(
        _mine_body,
        grid=(_NB,),
        in_specs=[
            pl.BlockSpec((_B, _FEAT), lambda i: (i, 0)),
            pl.BlockSpec((_FEAT, _EMB), lambda i: (0, 0)),
        ],
        out_specs=[
            pl.BlockSpec((_B, _EMB), lambda i: (i, 0)),
            pl.BlockSpec((_B, _EMB), lambda i: (i, 0)),
            pl.BlockSpec((_B, _EMB), lambda i: (i, 0)),
            pl.BlockSpec((_B, _AUG), lambda i: (i, 0)),
        ],
        out_shape=[
            jax.ShapeDtypeStruct((_N, _EMB), jnp.float32),
            jax.ShapeDtypeStruct((_N, _EMB), jnp.float32),
            jax.ShapeDtypeStruct((_N, _EMB), jnp.float32),
            jax.ShapeDtypeStruct((_N, _AUG), jnp.float32),
        ],
    )(feats, wp)


def _seg_body(table, ei3, out, src_v, dst_v, rows0, rows1, zbuf, acc,
              sem0, sem1):
    cid = lax.axis_index("c")
    sid = lax.axis_index("s")
    wid = sid * 2 + cid

    def zrow(i, carry):
        for j in range(_AUG // 16):
            zbuf[i, pl.ds(j * 16, 16)] = jnp.zeros((16,), jnp.float32)
        return carry
    lax.fori_loop(0, 128, zrow, 0)
    for r in range(_RPS // 128):
        pltpu.sync_copy(zbuf, acc.at[pl.ds(sid * _RPS + r * 128, 128)])
    plsc.subcore_barrier()

    pltpu.sync_copy(ei3.at[0, pl.ds(wid * _NCH, _NCH)], src_v)
    pltpu.sync_copy(ei3.at[1, pl.ds(wid * _NCH, _NCH)], dst_v)

    def gather(c, buf, sem):
        return pltpu.make_async_copy(table.at[src_v.at[c]], buf, sem)

    gather(0, rows0, sem0).start()
    gather(1, rows1, sem1).start()

    def pair(j, carry):
        c0 = j * 2
        for c, buf, sem in ((c0, rows0, sem0), (c0 + 1, rows1, sem1)):
            gather(c, buf, sem).wait()
            pltpu.sync_copy(buf, acc.at[dst_v.at[c]], add=True)

            @pl.when(c + 2 < _NCH)
            def _():
                gather(c + 2, buf, sem).start()
        return carry
    lax.fori_loop(0, _NCH // 2, pair, 0)

    plsc.subcore_barrier()
    pltpu.sync_copy(acc.at[pl.ds(sid * _RPS, _RPS)],
                    out.at[pl.ds(cid * _N + sid * _RPS, _RPS)])


def _segsum(aug, ei3):
    k = pl.kernel(
        _seg_body,
        out_type=jax.ShapeDtypeStruct((2 * _N, _AUG), jnp.float32),
        mesh=plsc.VectorSubcoreMesh(core_axis_name="c", subcore_axis_name="s"),
        scratch_types=[
            pltpu.VMEM((_NCH, _CH), jnp.int32),
            pltpu.VMEM((_NCH, _CH), jnp.int32),
            pltpu.VMEM((_CH, _AUG), jnp.float32),
            pltpu.VMEM((_CH, _AUG), jnp.float32),
            pltpu.VMEM((128, _AUG), jnp.float32),
            pltpu.VMEM_SHARED((_N, _AUG), jnp.float32),
            pltpu.SemaphoreType.DMA,
            pltpu.SemaphoreType.DMA,
        ],
        compiler_params=pltpu.CompilerParams(use_tc_tiling_on_sc=False),
    )
    return k(aug, ei3)


def _head_body(p0_ref, p1_ref, wm_ref, wc_ref, mg_ref, lg_ref):
    accv = p0_ref[...] + p1_ref[...]
    agg = accv[:, :_EMB]
    deg = accv[:, _EMB:_EMB + 1]
    mean = agg / jnp.maximum(deg, 1.0)
    mg = jnp.maximum(
        lax.dot_general(mean, wm_ref[...], (((1,), (0,)), ((), ())),
                        preferred_element_type=jnp.float32), 0.0)
    mg_ref[...] = mg
    lg_ref[...] = lax.dot_general(mg, wc_ref[...], (((1,), (0,)), ((), ())),
                                  preferred_element_type=jnp.float32)


def _head(partials, wm, wc):
    return pl.pallas_call(
        _head_body,
        grid=(_NB,),
        in_specs=[
            pl.BlockSpec((_B, _AUG), lambda i: (i, 0)),
            pl.BlockSpec((_B, _AUG), lambda i: (i + _NB, 0)),
            pl.BlockSpec((_MESH, _MESH), lambda i: (0, 0)),
            pl.BlockSpec((_MESH, _OUT), lambda i: (0, 0)),
        ],
        out_specs=[
            pl.BlockSpec((_B, _MESH), lambda i: (i, 0)),
            pl.BlockSpec((_B, _OUT), lambda i: (i, 0)),
        ],
        out_shape=[
            jax.ShapeDtypeStruct((_N, _MESH), jnp.float32),
            jax.ShapeDtypeStruct((_N, _OUT), jnp.float32),
        ],
    )(partials, partials, wm, wc)


def kernel(patch_feats, edge_index, W_patch, W_mesh, W_cls):
    sp, pos, neg, aug = _mine(patch_feats, W_patch)
    partials = _segsum(aug, edge_index.reshape(2, _E // _CH, _CH))
    mg, logits = _head(partials, W_mesh, W_cls)
    return (logits, mg, sp, pos, neg)
